# baseline (speedup 1.0000x reference)
import functools

import jax
import jax.numpy as jnp
import numpy as np
from jax import lax
from jax.experimental import pallas as pl
from jax.experimental.pallas import tpu as pltpu
from jax.experimental.pallas import tpu_sc as plsc

BATCH = 1024
SEQ = 200
EDIM = 128
ATTR = 8
NCLS = SEQ // ATTR
NROWS = BATCH * NCLS

_NC, _NS = 2, 16
_NW = _NC * _NS
_CHUNK = 80


def _sc_gather_fn(nrows):
    mesh = plsc.VectorSubcoreMesh(core_axis_name="c", subcore_axis_name="s")
    rpw = nrows // _NW
    nchunk = rpw // _CHUNK

    @functools.partial(
        pl.kernel,
        mesh=mesh,
        out_type=jax.ShapeDtypeStruct((nrows, EDIM), jnp.float32),
        scratch_types=[
            pltpu.VMEM((nchunk, _CHUNK), jnp.int32),
            pltpu.VMEM((rpw, EDIM), jnp.float32),
            pltpu.SemaphoreType.DMA,
        ],
    )
    def sc_gather(idx_hbm, table_hbm, out_hbm, idx_v, rows_v, sem):
        wid = lax.axis_index("s") * _NC + lax.axis_index("c")
        pltpu.sync_copy(idx_hbm.at[wid], idx_v)
        copies = [
            pltpu.async_copy(
                table_hbm.at[idx_v.at[j]],
                rows_v.at[pl.ds(j * _CHUNK, _CHUNK)],
                sem,
            )
            for j in range(nchunk)
        ]
        for c in copies:
            c.wait()
        pltpu.sync_copy(rows_v, out_hbm.at[pl.ds(wid * rpw, rpw)])

    return sc_gather


_BB = 32
_RESTART = 16


def _tc_body(xf_ref, g_ref, o_ref):
    w = xf_ref[...].astype(jnp.float32) * np.float32(np.pi)
    s = jnp.sin(w)
    c = jnp.cos(w)
    slist, clist = [s], [c]
    for k in range(1, EDIM // 2):
        if k % _RESTART == 0:
            ph = w * np.float32(2.0 ** k)
            s = jnp.sin(ph)
            c = jnp.cos(ph)
        else:
            s, c = 2.0 * s * c, c * c - s * s
        slist.append(s)
        clist.append(c)
    S = jnp.stack(slist + clist, axis=0)
    r0 = lax.broadcasted_iota(jnp.int32, (EDIM, EDIM), 0)
    r1 = lax.broadcasted_iota(jnp.int32, (EDIM, EDIM), 1)
    ident = (r0 == r1).astype(jnp.float32)
    O = lax.dot_general(
        S, ident, (((0,), (0,)), ((), ())),
        precision=lax.Precision.DEFAULT,
    )
    O5 = O.reshape(_BB, NCLS, ATTR, EDIM)
    sub = lax.broadcasted_iota(jnp.int32, (_BB, NCLS, ATTR, EDIM), 2)
    o_ref[...] = jnp.where(sub == 0, g_ref[...], O5)


_HBATCH = BATCH // 2
_HGRID = _HBATCH // _BB


def kernel(x, E_class):
    hc = _HBATCH * NCLS
    sc = _sc_gather_fn(hc)
    idx1 = x[:_HBATCH, ::ATTR].reshape(_NW, hc // _NW // _CHUNK, _CHUNK)
    idx2 = x[_HBATCH:, ::ATTR].reshape(_NW, hc // _NW // _CHUNK, _CHUNK)
    g1 = sc(idx1, E_class).reshape(_HBATCH, NCLS, 1, EDIM)
    g2 = sc(idx2, E_class).reshape(_HBATCH, NCLS, 1, EDIM)

    out_shape = jax.ShapeDtypeStruct((BATCH, NCLS, ATTR, EDIM), jnp.float32)
    o1 = pl.pallas_call(
        _tc_body,
        grid=(_HGRID,),
        in_specs=[
            pl.BlockSpec((_BB, SEQ), lambda i: (i, 0)),
            pl.BlockSpec((_BB, NCLS, 1, EDIM), lambda i: (i, 0, 0, 0)),
        ],
        out_specs=pl.BlockSpec((_BB, NCLS, ATTR, EDIM), lambda i: (i, 0, 0, 0)),
        out_shape=out_shape,
    )(x, g1)

    def _tc_body2(xf_ref, g_ref, prev_ref, o_ref):
        _tc_body(xf_ref, g_ref, o_ref)

    out4 = pl.pallas_call(
        _tc_body2,
        grid=(_HGRID,),
        in_specs=[
            pl.BlockSpec((_BB, SEQ), lambda i: (i + _HGRID, 0)),
            pl.BlockSpec((_BB, NCLS, 1, EDIM), lambda i: (i, 0, 0, 0)),
            pl.BlockSpec(memory_space=pltpu.MemorySpace.HBM),
        ],
        out_specs=pl.BlockSpec((_BB, NCLS, ATTR, EDIM), lambda i: (i + _HGRID, 0, 0, 0)),
        out_shape=out_shape,
        input_output_aliases={2: 0},
    )(x, g2, o1)

    return out4.reshape(BATCH, SEQ, EDIM)
